# Initial kernel scaffold; baseline (speedup 1.0000x reference)
import jax
import jax.numpy as jnp
from jax.experimental import pallas as pl


def kernel(xyz, features, params):
    raise NotImplementedError("write your pallas kernel here")


# speedup vs baseline: 27.3745x; 27.3745x over previous
import functools

import jax
import jax.numpy as jnp
import numpy as np
from jax import lax
from jax.experimental import pallas as pl
from jax.experimental.pallas import tpu as pltpu

_BN = np.float32(1.0 / np.sqrt(1.0 + 1e-5))
_DIMS = [32, 64, 128, 256]
_K = 16
_NQ = 256

_INTERPRET = False


def _mlp(x, wt_ref, b_ref):
    y = jnp.dot(x, wt_ref[...], preferred_element_type=jnp.float32) + b_ref[...]
    return jnp.maximum(y * _BN, 0.0)


def _knn_agg(qt, cand_t, cand_r, ew1, eb1, ew2, eb2):
    ns = cand_t.shape[1]
    sqc = jnp.sum(cand_t * cand_t, axis=0, keepdims=True)
    sqq = jnp.sum(qt * qt, axis=1, keepdims=True)
    d2 = sqq + sqc - 2.0 * jnp.dot(qt, cand_t, preferred_element_type=jnp.float32)
    col = lax.broadcasted_iota(jnp.int32, (_NQ, ns), 1)
    sp_parts = []
    for _ in range(_K):
        rmin = jnp.min(d2, axis=1, keepdims=True)
        sel = jnp.min(jnp.where(d2 == rmin, col, ns), axis=1, keepdims=True)
        hit = col == sel
        onehot = hit.astype(jnp.float32)
        d2 = jnp.where(hit, jnp.inf, d2)
        nbr = jnp.dot(onehot, cand_r, preferred_element_type=jnp.float32)
        rel = nbr - qt
        dist = jnp.sqrt(jnp.sum(rel * rel, axis=1, keepdims=True) + 1e-12)
        sp_parts.append(jnp.concatenate([rel, dist, qt, nbr], axis=1))
    sp = jnp.concatenate(sp_parts, axis=0)
    h = _mlp(sp, ew1, eb1)
    h = _mlp(h, ew2, eb2)
    agg = h[0:_NQ]
    for k in range(1, _K):
        agg = jnp.maximum(agg, h[k * _NQ:(k + 1) * _NQ])
    return agg


def _tc_body(*refs):
    (c0t, c0r, c1t, c1r, c2t, c2r, fin, emb_wt, emb_b), rest = refs[:9], refs[9:]
    out_ref = rest[-1]
    wrefs = rest[:-1]
    qt = c2r[0]
    x = _mlp(fin[0], emb_wt, emb_b)
    cands = [(c0t, c0r), (c1t, c1r), (c2t, c2r)]
    for s in range(3):
        (w_e1, b_e1, w_e2, b_e2, w_a1, b_a1, w_a2, b_a2, w_ac, b_ac,
         w_o1, b_o1, w_o2, b_o2, w_sh, b_sh) = wrefs[s * 16:(s + 1) * 16]
        ct, cr = cands[s]
        agg = _knn_agg(qt, ct[0], cr[0], w_e1, b_e1, w_e2, b_e2)
        att = jnp.concatenate([x, agg], axis=1)
        att = _mlp(att, w_a1, b_a1)
        att = _mlp(att, w_a2, b_a2)
        att = jax.nn.sigmoid(
            jnp.dot(att, w_ac[...], preferred_element_type=jnp.float32) + b_ac[...])
        weighted = x * att
        o = _mlp(weighted, w_o1, b_o1)
        o = _mlp(o, w_o2, b_o2)
        sh = (jnp.dot(x, w_sh[...], preferred_element_type=jnp.float32)
              + b_sh[...]) * _BN
        x = jnp.maximum(o + sh, 0.0)
    out_ref[0] = x.T


def _wt(wb):
    w, b = wb
    return [w.T, b[None, :]]


def kernel(xyz, features, params):
    B, N, _ = xyz.shape
    f32 = jnp.float32

    key = jax.random.key(1234)
    cur, sel = N, []
    for i in range(3):
        S = max(1, int(cur * 0.25))
        r = jax.random.uniform(jax.random.fold_in(key, i), (B, cur))
        sel.append(jnp.argsort(r, axis=1)[:, :S])
        cur = S
    a0, a1, a2 = sel
    i0 = a0
    i1 = jnp.take_along_axis(i0, a1, axis=1)
    i2 = jnp.take_along_axis(i1, a2, axis=1)

    def rows(idx):
        return jnp.take_along_axis(xyz, idx[:, :, None], axis=1)

    c0r, c1r, c2r = rows(i0), rows(i1), rows(i2)
    fin = jnp.take_along_axis(
        jnp.transpose(features, (0, 2, 1)), i2[:, :, None], axis=1)
    c0t = jnp.transpose(c0r, (0, 2, 1))
    c1t = jnp.transpose(c1r, (0, 2, 1))
    c2t = jnp.transpose(c2r, (0, 2, 1))

    ws = _wt(params['embedding'][0])
    for p in params['lfa']:
        for wb in p['enc']:
            ws += _wt(wb)
        for wb in p['att_mlp']:
            ws += _wt(wb)
        ws += _wt(p['att_conv'])
        for wb in p['out']:
            ws += _wt(wb)
        ws += _wt(p['short'])

    data = [c0t, c0r, c1t, c1r, c2t, c2r, fin]
    d_specs = [pl.BlockSpec((1,) + d.shape[1:], lambda b: (b, 0, 0)) for d in data]
    w_specs = [pl.BlockSpec(w.shape, lambda b: (0, 0)) for w in ws]

    out = pl.pallas_call(
        _tc_body,
        grid=(B,),
        in_specs=d_specs + w_specs,
        out_specs=pl.BlockSpec((1, _DIMS[-1], _NQ), lambda b: (b, 0, 0)),
        out_shape=jax.ShapeDtypeStruct((B, _DIMS[-1], _NQ), f32),
        interpret=_INTERPRET,
    )(*data, *ws)
    return out


# speedup vs baseline: 28.8275x; 1.0531x over previous
import functools

import jax
import jax.numpy as jnp
import numpy as np
from jax import lax
from jax.experimental import pallas as pl
from jax.experimental.pallas import tpu as pltpu
from jax.experimental.pallas import tpu_sc as plsc

_BN = np.float32(1.0 / np.sqrt(1.0 + 1e-5))
_DIMS = [32, 64, 128, 256]
_K = 16
_NQ = 256

_INTERPRET = False


def _mlp(x, wt_ref, b_ref):
    y = jnp.dot(x, wt_ref[...], preferred_element_type=jnp.float32) + b_ref[...]
    return jnp.maximum(y * _BN, 0.0)


def _knn_agg(qt, cand_t, cand_r, ew1, eb1, ew2, eb2):
    ns = cand_t.shape[1]
    sqc = jnp.sum(cand_t * cand_t, axis=0, keepdims=True)
    sqq = jnp.sum(qt * qt, axis=1, keepdims=True)
    d2 = sqq + sqc - 2.0 * jnp.dot(qt, cand_t, preferred_element_type=jnp.float32)
    col = lax.broadcasted_iota(jnp.int32, (_NQ, ns), 1)
    sp_parts = []
    for _ in range(_K):
        rmin = jnp.min(d2, axis=1, keepdims=True)
        sel = jnp.min(jnp.where(d2 == rmin, col, ns), axis=1, keepdims=True)
        hit = col == sel
        onehot = hit.astype(jnp.float32)
        d2 = jnp.where(hit, jnp.inf, d2)
        nbr = jnp.dot(onehot, cand_r, preferred_element_type=jnp.float32)
        rel = nbr - qt
        dist = jnp.sqrt(jnp.sum(rel * rel, axis=1, keepdims=True) + 1e-12)
        sp_parts.append(jnp.concatenate([rel, dist, qt, nbr], axis=1))
    sp = jnp.concatenate(sp_parts, axis=0)
    h = _mlp(sp, ew1, eb1)
    h = _mlp(h, ew2, eb2)
    agg = h[0:_NQ]
    for k in range(1, _K):
        agg = jnp.maximum(agg, h[k * _NQ:(k + 1) * _NQ])
    return agg


def _tc_body(*refs):
    (c0t, c0r, c1t, c1r, c2t, c2r, fin, emb_wt, emb_b), rest = refs[:9], refs[9:]
    out_ref = rest[-1]
    wrefs = rest[:-1]
    qt = c2r[0]
    x = _mlp(fin[0], emb_wt, emb_b)
    cands = [(c0t, c0r), (c1t, c1r), (c2t, c2r)]
    for s in range(3):
        (w_e1, b_e1, w_e2, b_e2, w_a1, b_a1, w_a2, b_a2, w_ac, b_ac,
         w_o1, b_o1, w_o2, b_o2, w_sh, b_sh) = wrefs[s * 16:(s + 1) * 16]
        ct, cr = cands[s]
        agg = _knn_agg(qt, ct[0], cr[0], w_e1, b_e1, w_e2, b_e2)
        att = jnp.concatenate([x, agg], axis=1)
        att = _mlp(att, w_a1, b_a1)
        att = _mlp(att, w_a2, b_a2)
        att = jax.nn.sigmoid(
            jnp.dot(att, w_ac[...], preferred_element_type=jnp.float32) + b_ac[...])
        weighted = x * att
        o = _mlp(weighted, w_o1, b_o1)
        o = _mlp(o, w_o2, b_o2)
        sh = (jnp.dot(x, w_sh[...], preferred_element_type=jnp.float32)
              + b_sh[...]) * _BN
        x = jnp.maximum(o + sh, 0.0)
    out_ref[0] = x.T


def _sc_gather(table, idx2d, n_chunks_total, chunk, width):
    info = plsc.get_sparse_core_info()
    nw = info.num_cores * info.num_subcores
    per_w = n_chunks_total // nw
    mesh = plsc.VectorSubcoreMesh(core_axis_name="c", subcore_axis_name="s")

    @functools.partial(
        pl.kernel,
        mesh=mesh,
        out_type=jax.ShapeDtypeStruct((n_chunks_total, chunk, width), jnp.float32),
        scratch_types=[
            pltpu.VMEM((chunk,), jnp.int32),
            pltpu.VMEM((chunk, width), jnp.float32),
            pltpu.SemaphoreType.DMA,
        ],
    )
    def run(table_hbm, idx_hbm, out_hbm, idx_v, rows_v, sem):
        wid = lax.axis_index("s") * info.num_cores + lax.axis_index("c")
        for j in range(per_w):
            r = wid * per_w + j
            pltpu.sync_copy(idx_hbm.at[r], idx_v)
            pltpu.async_copy(table_hbm.at[idx_v], rows_v, sem).wait()
            pltpu.sync_copy(rows_v, out_hbm.at[r])

    return run(table, idx2d)


def _wt(wb):
    w, b = wb
    return [w.T, b[None, :]]


def kernel(xyz, features, params):
    B, N, _ = xyz.shape
    f32 = jnp.float32

    key = jax.random.key(1234)
    cur, sel = N, []
    for i in range(3):
        S = max(1, int(cur * 0.25))
        r = jax.random.uniform(jax.random.fold_in(key, i), (B, cur))
        sel.append(jnp.argsort(r, axis=1)[:, :S])
        cur = S
    a0, a1, a2 = sel
    i0 = a0
    i1 = jnp.take_along_axis(i0, a1, axis=1)
    i2 = jnp.take_along_axis(i1, a2, axis=1)

    s0, s1, s2 = i0.shape[1], i1.shape[1], i2.shape[1]
    per_b = s0 + s1 + s2
    width = 128
    table = jnp.concatenate(
        [xyz, jnp.transpose(features, (0, 2, 1)),
         jnp.zeros((B, N, width - 6), f32)], axis=2).reshape(B * N, width)
    idx_all = (jnp.concatenate([i0, i1, i2], axis=1)
               + (jnp.arange(B, dtype=i0.dtype) * N)[:, None]).reshape(-1)
    total = B * per_b
    nw = 32
    per_w = total // nw
    n_chunks = 1
    while per_w % n_chunks or per_w // n_chunks > 128:
        n_chunks += 1
    chunk = per_w // n_chunks
    g = _sc_gather(table, idx_all.reshape(total // chunk, chunk).astype(jnp.int32),
                   total // chunk, chunk, width)
    g = g.reshape(B, per_b, width)
    c0r = g[:, :s0, :3]
    c1r = g[:, s0:s0 + s1, :3]
    c2r = g[:, s0 + s1:, :3]
    fin = g[:, s0 + s1:, 3:6]
    c0t = jnp.transpose(c0r, (0, 2, 1))
    c1t = jnp.transpose(c1r, (0, 2, 1))
    c2t = jnp.transpose(c2r, (0, 2, 1))

    ws = _wt(params['embedding'][0])
    for p in params['lfa']:
        for wb in p['enc']:
            ws += _wt(wb)
        for wb in p['att_mlp']:
            ws += _wt(wb)
        ws += _wt(p['att_conv'])
        for wb in p['out']:
            ws += _wt(wb)
        ws += _wt(p['short'])

    data = [c0t, c0r, c1t, c1r, c2t, c2r, fin]
    d_specs = [pl.BlockSpec((1,) + d.shape[1:], lambda b: (b, 0, 0)) for d in data]
    w_specs = [pl.BlockSpec(w.shape, lambda b: (0, 0)) for w in ws]

    out = pl.pallas_call(
        _tc_body,
        grid=(B,),
        in_specs=d_specs + w_specs,
        out_specs=pl.BlockSpec((1, _DIMS[-1], _NQ), lambda b: (b, 0, 0)),
        out_shape=jax.ShapeDtypeStruct((B, _DIMS[-1], _NQ), f32),
        interpret=_INTERPRET,
    )(*data, *ws)
    return out


# speedup vs baseline: 48.4788x; 1.6817x over previous
import functools

import jax
import jax.numpy as jnp
import numpy as np
from jax import lax
from jax.experimental import pallas as pl
from jax.experimental.pallas import tpu as pltpu
from jax.experimental.pallas import tpu_sc as plsc

_BN = np.float32(1.0 / np.sqrt(1.0 + 1e-5))
_DIMS = [32, 64, 128, 256]
_K = 16
_NQ = 256

_INTERPRET = False


def _mlp(x, wt_ref, b_ref):
    y = jnp.dot(x, wt_ref[...], preferred_element_type=jnp.float32) + b_ref[...]
    return jnp.maximum(y * _BN, 0.0)


def _knn_agg(qt, cand_t, cand_r, ew1, eb1, ew2, eb2):
    ns = cand_t.shape[1]
    sqc = jnp.sum(cand_t * cand_t, axis=0, keepdims=True)
    sqq = jnp.sum(qt * qt, axis=1, keepdims=True)
    d2 = sqq + sqc - 2.0 * jnp.dot(qt, cand_t, preferred_element_type=jnp.float32)
    col = lax.broadcasted_iota(jnp.int32, (_NQ, ns), 1)
    sp_parts = []
    for _ in range(_K):
        rmin = jnp.min(d2, axis=1, keepdims=True)
        sel = jnp.min(jnp.where(d2 == rmin, col, ns), axis=1, keepdims=True)
        hit = col == sel
        onehot = hit.astype(jnp.float32)
        d2 = jnp.where(hit, jnp.inf, d2)
        nbr = jnp.dot(onehot, cand_r, preferred_element_type=jnp.float32)
        rel = nbr - qt
        dist = jnp.sqrt(jnp.sum(rel * rel, axis=1, keepdims=True) + 1e-12)
        sp_parts.append(jnp.concatenate([rel, dist, qt, nbr], axis=1))
    sp = jnp.concatenate(sp_parts, axis=0)
    h = _mlp(sp, ew1, eb1)
    h = _mlp(h, ew2, eb2)
    agg = h[0:_NQ]
    for k in range(1, _K):
        agg = jnp.maximum(agg, h[k * _NQ:(k + 1) * _NQ])
    return agg


def _tc_body(*refs):
    (c0t, c0r, c1t, c1r, c2t, c2r, fin, emb_wt, emb_b), rest = refs[:9], refs[9:]
    out_ref = rest[-1]
    wrefs = rest[:-1]
    qt = c2r[0]
    x = _mlp(fin[0], emb_wt, emb_b)
    cands = [(c0t, c0r), (c1t, c1r), (c2t, c2r)]
    for s in range(3):
        (w_e1, b_e1, w_e2, b_e2, w_a1, b_a1, w_a2, b_a2, w_ac, b_ac,
         w_o1, b_o1, w_o2, b_o2, w_sh, b_sh) = wrefs[s * 16:(s + 1) * 16]
        ct, cr = cands[s]
        agg = _knn_agg(qt, ct[0], cr[0], w_e1, b_e1, w_e2, b_e2)
        att = jnp.concatenate([x, agg], axis=1)
        att = _mlp(att, w_a1, b_a1)
        att = _mlp(att, w_a2, b_a2)
        att = jax.nn.sigmoid(
            jnp.dot(att, w_ac[...], preferred_element_type=jnp.float32) + b_ac[...])
        weighted = x * att
        o = _mlp(weighted, w_o1, b_o1)
        o = _mlp(o, w_o2, b_o2)
        sh = (jnp.dot(x, w_sh[...], preferred_element_type=jnp.float32)
              + b_sh[...]) * _BN
        x = jnp.maximum(o + sh, 0.0)
    out_ref[0] = x.T


def _sc_gather(table, idx2d, n_chunks_total, chunk, width):
    info = plsc.get_sparse_core_info()
    nw = info.num_cores * info.num_subcores
    per_w = n_chunks_total // nw
    mesh = plsc.VectorSubcoreMesh(core_axis_name="c", subcore_axis_name="s")

    @functools.partial(
        pl.kernel,
        mesh=mesh,
        out_type=jax.ShapeDtypeStruct((n_chunks_total, chunk, width), jnp.float32),
        scratch_types=[
            pltpu.VMEM((chunk,), jnp.int32),
            pltpu.VMEM((chunk, width), jnp.float32),
            pltpu.SemaphoreType.DMA,
        ],
    )
    def run(table_hbm, idx_hbm, out_hbm, idx_v, rows_v, sem):
        wid = lax.axis_index("s") * info.num_cores + lax.axis_index("c")
        for j in range(per_w):
            r = wid * per_w + j
            pltpu.sync_copy(idx_hbm.at[r], idx_v)
            pltpu.async_copy(table_hbm.at[idx_v], rows_v, sem).wait()
            pltpu.sync_copy(rows_v, out_hbm.at[r])

    return run(table, idx2d)


def _wt(wb):
    w, b = wb
    return [w.T, b[None, :]]


@functools.lru_cache(maxsize=None)
def _decim_indices(B, N):
    cur, sel = N, []
    with jax.ensure_compile_time_eval():
        key = jax.random.key(1234)
        for i in range(3):
            S = max(1, int(cur * 0.25))
            r = np.asarray(
                jax.random.uniform(jax.random.fold_in(key, i), (B, cur)))
            sel.append(np.argsort(r, axis=1, kind='stable')[:, :S])
            cur = S
    a0, a1, a2 = sel
    i0 = a0
    i1 = np.take_along_axis(i0, a1, axis=1)
    i2 = np.take_along_axis(i1, a2, axis=1)
    return i0.astype(np.int32), i1.astype(np.int32), i2.astype(np.int32)


def kernel(xyz, features, params):
    B, N, _ = xyz.shape
    f32 = jnp.float32
    i0, i1, i2 = _decim_indices(B, N)

    s0, s1, s2 = i0.shape[1], i1.shape[1], i2.shape[1]
    per_b = s0 + s1 + s2
    width = 128
    table = jnp.concatenate(
        [xyz, jnp.transpose(features, (0, 2, 1)),
         jnp.zeros((B, N, width - 6), f32)], axis=2).reshape(B * N, width)
    idx_all = (np.concatenate([i0, i1, i2], axis=1)
               + (np.arange(B, dtype=np.int32) * N)[:, None]).reshape(-1)
    total = B * per_b
    nw = 32
    per_w = total // nw
    n_chunks = 1
    while per_w % n_chunks or per_w // n_chunks > 128:
        n_chunks += 1
    chunk = per_w // n_chunks
    g = _sc_gather(table,
                   jnp.asarray(idx_all.reshape(total // chunk, chunk)),
                   total // chunk, chunk, width)
    g = g.reshape(B, per_b, width)
    c0r = g[:, :s0, :3]
    c1r = g[:, s0:s0 + s1, :3]
    c2r = g[:, s0 + s1:, :3]
    fin = g[:, s0 + s1:, 3:6]
    c0t = jnp.transpose(c0r, (0, 2, 1))
    c1t = jnp.transpose(c1r, (0, 2, 1))
    c2t = jnp.transpose(c2r, (0, 2, 1))

    ws = _wt(params['embedding'][0])
    for p in params['lfa']:
        for wb in p['enc']:
            ws += _wt(wb)
        for wb in p['att_mlp']:
            ws += _wt(wb)
        ws += _wt(p['att_conv'])
        for wb in p['out']:
            ws += _wt(wb)
        ws += _wt(p['short'])

    data = [c0t, c0r, c1t, c1r, c2t, c2r, fin]
    d_specs = [pl.BlockSpec((1,) + d.shape[1:], lambda b: (b, 0, 0)) for d in data]
    w_specs = [pl.BlockSpec(w.shape, lambda b: (0, 0)) for w in ws]

    out = pl.pallas_call(
        _tc_body,
        grid=(B,),
        in_specs=d_specs + w_specs,
        out_specs=pl.BlockSpec((1, _DIMS[-1], _NQ), lambda b: (b, 0, 0)),
        out_shape=jax.ShapeDtypeStruct((B, _DIMS[-1], _NQ), f32),
        interpret=_INTERPRET,
    )(*data, *ws)
    return out


# speedup vs baseline: 54.2119x; 1.1183x over previous
import functools

import jax
import jax.numpy as jnp
import numpy as np
from jax import lax
from jax.experimental import pallas as pl
from jax.experimental.pallas import tpu as pltpu
from jax.experimental.pallas import tpu_sc as plsc

_BN = np.float32(1.0 / np.sqrt(1.0 + 1e-5))
_DIMS = [32, 64, 128, 256]
_K = 16
_NQ = 256

_INTERPRET = False


def _mlp(x, wt_ref, b_ref):
    y = jnp.dot(x, wt_ref[...], preferred_element_type=jnp.float32) + b_ref[...]
    return jnp.maximum(y * _BN, 0.0)


def _knn_agg(qt, cand_t, wa, wb, wd, eb1, ew2, eb2):
    ns = cand_t.shape[1]
    sqc = jnp.sum(cand_t * cand_t, axis=0, keepdims=True)
    sqq = jnp.sum(qt * qt, axis=1, keepdims=True)
    d2 = sqq + sqc - 2.0 * jnp.dot(qt, cand_t, preferred_element_type=jnp.float32)
    pt = lax.dot_general(wa[...], cand_t, (((0,), (0,)), ((), ())),
                         preferred_element_type=jnp.float32)
    qbase = jnp.dot(qt, wb[...], preferred_element_type=jnp.float32) + eb1[...]
    col = lax.broadcasted_iota(jnp.int32, (_NQ, ns), 1)
    agg = None
    for _ in range(_K):
        rmin = jnp.min(d2, axis=1, keepdims=True)
        sel = jnp.min(jnp.where(d2 == rmin, col, ns), axis=1, keepdims=True)
        hit = col == sel
        d2 = jnp.where(hit, jnp.inf, d2)
        dist = jnp.sqrt(jnp.maximum(rmin, 0.0) + 1e-12)
        selp = lax.dot_general(hit.astype(jnp.float32), pt,
                               (((1,), (1,)), ((), ())),
                               preferred_element_type=jnp.float32)
        enc = jnp.maximum((selp + qbase + dist * wd[...]) * _BN, 0.0)
        enc = _mlp(enc, ew2, eb2)
        agg = enc if agg is None else jnp.maximum(agg, enc)
    return agg


def _tc_body(*refs):
    (c0t, c1t, c2t, qtr, fin, emb_wt, emb_b), rest = refs[:7], refs[7:]
    out_ref = rest[-1]
    wrefs = rest[:-1]
    qt = qtr[0]
    x = _mlp(fin[0], emb_wt, emb_b)
    cands = [c0t, c1t, c2t]
    for s in range(3):
        (w_a, w_b, w_d, b_e1, w_e2, b_e2, w_a1, b_a1, w_a2, b_a2, w_ac, b_ac,
         w_o1, b_o1, w_o2, b_o2, w_sh, b_sh) = wrefs[s * 18:(s + 1) * 18]
        agg = _knn_agg(qt, cands[s][0], w_a, w_b, w_d, b_e1, w_e2, b_e2)
        att = jnp.concatenate([x, agg], axis=1)
        att = _mlp(att, w_a1, b_a1)
        att = _mlp(att, w_a2, b_a2)
        att = jax.nn.sigmoid(
            jnp.dot(att, w_ac[...], preferred_element_type=jnp.float32) + b_ac[...])
        weighted = x * att
        o = _mlp(weighted, w_o1, b_o1)
        o = _mlp(o, w_o2, b_o2)
        sh = (jnp.dot(x, w_sh[...], preferred_element_type=jnp.float32)
              + b_sh[...]) * _BN
        x = jnp.maximum(o + sh, 0.0)
    out_ref[0] = x.T


def _sc_gather(table, idx2d, n_chunks_total, chunk, width):
    info = plsc.get_sparse_core_info()
    nw = info.num_cores * info.num_subcores
    per_w = n_chunks_total // nw
    mesh = plsc.VectorSubcoreMesh(core_axis_name="c", subcore_axis_name="s")

    @functools.partial(
        pl.kernel,
        mesh=mesh,
        out_type=jax.ShapeDtypeStruct((n_chunks_total, chunk, width), jnp.float32),
        scratch_types=[
            pltpu.VMEM((chunk,), jnp.int32),
            pltpu.VMEM((chunk, width), jnp.float32),
            pltpu.SemaphoreType.DMA,
        ],
    )
    def run(table_hbm, idx_hbm, out_hbm, idx_v, rows_v, sem):
        wid = lax.axis_index("s") * info.num_cores + lax.axis_index("c")
        for j in range(per_w):
            r = wid * per_w + j
            pltpu.sync_copy(idx_hbm.at[r], idx_v)
            pltpu.async_copy(table_hbm.at[idx_v], rows_v, sem).wait()
            pltpu.sync_copy(rows_v, out_hbm.at[r])

    return run(table, idx2d)


def _wt(wb):
    w, b = wb
    return [w.T, b[None, :]]


@functools.lru_cache(maxsize=None)
def _decim_indices(B, N):
    cur, sel = N, []
    with jax.ensure_compile_time_eval():
        key = jax.random.key(1234)
        for i in range(3):
            S = max(1, int(cur * 0.25))
            r = np.asarray(
                jax.random.uniform(jax.random.fold_in(key, i), (B, cur)))
            sel.append(np.argsort(r, axis=1, kind='stable')[:, :S])
            cur = S
    a0, a1, a2 = sel
    i0 = a0
    i1 = np.take_along_axis(i0, a1, axis=1)
    i2 = np.take_along_axis(i1, a2, axis=1)
    return i0.astype(np.int32), i1.astype(np.int32), i2.astype(np.int32)


def kernel(xyz, features, params):
    B, N, _ = xyz.shape
    f32 = jnp.float32
    i0, i1, i2 = _decim_indices(B, N)

    s0, s1, s2 = i0.shape[1], i1.shape[1], i2.shape[1]
    per_b = s0 + s1 + s2
    width = 128
    table = jnp.concatenate(
        [xyz, jnp.transpose(features, (0, 2, 1)),
         jnp.zeros((B, N, width - 6), f32)], axis=2).reshape(B * N, width)
    idx_all = (np.concatenate([i0, i1, i2], axis=1)
               + (np.arange(B, dtype=np.int32) * N)[:, None]).reshape(-1)
    total = B * per_b
    nw = 32
    per_w = total // nw
    n_chunks = 1
    while per_w % n_chunks or per_w // n_chunks > 128:
        n_chunks += 1
    chunk = per_w // n_chunks
    g = _sc_gather(table,
                   jnp.asarray(idx_all.reshape(total // chunk, chunk)),
                   total // chunk, chunk, width)
    g = g.reshape(B, per_b, width)
    qtr = g[:, s0 + s1:, :3]
    fin = g[:, s0 + s1:, 3:6]
    gt = jnp.transpose(g[:, :, :3], (0, 2, 1))
    c0t = gt[:, :, :s0]
    c1t = gt[:, :, s0:s0 + s1]
    c2t = gt[:, :, s0 + s1:]
    ws = _wt(params['embedding'][0])
    for p in params['lfa']:
        w1, b1 = p['enc'][0]
        w1t = w1.T
        ws += [w1t[0:3] + w1t[7:10], w1t[4:7] - w1t[0:3], w1t[3:4], b1[None, :]]
        ws += _wt(p['enc'][1])
        for wb in p['att_mlp']:
            ws += _wt(wb)
        ws += _wt(p['att_conv'])
        for wb in p['out']:
            ws += _wt(wb)
        ws += _wt(p['short'])

    data = [c0t, c1t, c2t, qtr, fin]
    d_specs = [pl.BlockSpec((1,) + d.shape[1:], lambda b: (b, 0, 0)) for d in data]
    w_specs = [pl.BlockSpec(w.shape, lambda b: (0, 0)) for w in ws]

    out = pl.pallas_call(
        _tc_body,
        grid=(B,),
        in_specs=d_specs + w_specs,
        out_specs=pl.BlockSpec((1, _DIMS[-1], _NQ), lambda b: (b, 0, 0)),
        out_shape=jax.ShapeDtypeStruct((B, _DIMS[-1], _NQ), f32),
        interpret=_INTERPRET,
    )(*data, *ws)
    return out


# speedup vs baseline: 55.5286x; 1.0243x over previous
import functools

import jax
import jax.numpy as jnp
import numpy as np
from jax import lax
from jax.experimental import pallas as pl
from jax.experimental.pallas import tpu as pltpu
from jax.experimental.pallas import tpu_sc as plsc

_BN = np.float32(1.0 / np.sqrt(1.0 + 1e-5))
_DIMS = [32, 64, 128, 256]
_K = 16
_NQ = 256

_INTERPRET = False


def _mlp(x, wt_ref, b_ref):
    y = jnp.dot(x, wt_ref[...], preferred_element_type=jnp.float32) + b_ref[...]
    return jnp.maximum(y * _BN, 0.0)


def _knn_agg(qt, cand_t, wa, wb, wd, eb1, ew2, eb2):
    ns = cand_t.shape[1]
    sqc = jnp.sum(cand_t * cand_t, axis=0, keepdims=True)
    sqq = jnp.sum(qt * qt, axis=1, keepdims=True)
    d2 = sqq + sqc - 2.0 * jnp.dot(qt, cand_t, preferred_element_type=jnp.float32)
    pt = lax.dot_general(wa[...], cand_t, (((0,), (0,)), ((), ())),
                         preferred_element_type=jnp.float32)
    qbase = jnp.dot(qt, wb[...], preferred_element_type=jnp.float32) + eb1[...]
    col = lax.broadcasted_iota(jnp.int32, (_NQ, ns), 1)
    agg = None
    for k in range(_K):
        rmin = jnp.min(d2, axis=1, keepdims=True)
        tmp = jnp.where(d2 == rmin, col, ns)
        sel = jnp.min(tmp, axis=1, keepdims=True)
        hit = tmp == sel
        if k < _K - 1:
            d2 = jnp.where(hit, jnp.inf, d2)
        dist = jnp.sqrt(jnp.maximum(rmin, 0.0) + 1e-12)
        selp = lax.dot_general(hit.astype(jnp.float32), pt,
                               (((1,), (1,)), ((), ())),
                               preferred_element_type=jnp.float32)
        enc = jnp.maximum((selp + qbase + dist * wd[...]) * _BN, 0.0)
        enc = _mlp(enc, ew2, eb2)
        agg = enc if agg is None else jnp.maximum(agg, enc)
    return agg


def _tc_body(*refs):
    (c0t, c1t, c2t, qtr, fin, emb_wt, emb_b), rest = refs[:7], refs[7:]
    out_ref = rest[-1]
    wrefs = rest[:-1]
    qt = qtr[0]
    x = _mlp(fin[0], emb_wt, emb_b)
    cands = [c0t, c1t, c2t]
    for s in range(3):
        (w_a, w_b, w_d, b_e1, w_e2, b_e2, w_a1, b_a1, w_a2, b_a2, w_ac, b_ac,
         w_o1, b_o1, w_o2, b_o2, w_sh, b_sh) = wrefs[s * 18:(s + 1) * 18]
        agg = _knn_agg(qt, cands[s][0], w_a, w_b, w_d, b_e1, w_e2, b_e2)
        att = jnp.concatenate([x, agg], axis=1)
        att = _mlp(att, w_a1, b_a1)
        att = _mlp(att, w_a2, b_a2)
        att = jax.nn.sigmoid(
            jnp.dot(att, w_ac[...], preferred_element_type=jnp.float32) + b_ac[...])
        weighted = x * att
        o = _mlp(weighted, w_o1, b_o1)
        o = _mlp(o, w_o2, b_o2)
        sh = (jnp.dot(x, w_sh[...], preferred_element_type=jnp.float32)
              + b_sh[...]) * _BN
        x = jnp.maximum(o + sh, 0.0)
    out_ref[0] = x.T


def _sc_gather(table, idx2d, n_chunks_total, chunk, width):
    info = plsc.get_sparse_core_info()
    nw = info.num_cores * info.num_subcores
    per_w = n_chunks_total // nw
    mesh = plsc.VectorSubcoreMesh(core_axis_name="c", subcore_axis_name="s")

    @functools.partial(
        pl.kernel,
        mesh=mesh,
        out_type=jax.ShapeDtypeStruct((n_chunks_total, chunk, width), jnp.float32),
        scratch_types=[
            pltpu.VMEM((chunk,), jnp.int32),
            pltpu.VMEM((chunk, width), jnp.float32),
            pltpu.SemaphoreType.DMA,
        ],
        compiler_params=pltpu.CompilerParams(use_tc_tiling_on_sc=False),
    )
    def run(table_hbm, idx_hbm, out_hbm, idx_v, rows_v, sem):
        wid = lax.axis_index("s") * info.num_cores + lax.axis_index("c")
        for j in range(per_w):
            r = wid * per_w + j
            pltpu.sync_copy(idx_hbm.at[r], idx_v)
            pltpu.async_copy(table_hbm.at[idx_v], rows_v, sem).wait()
            pltpu.sync_copy(rows_v, out_hbm.at[r])

    return run(table, idx2d)


def _wt(wb):
    w, b = wb
    return [w.T, b[None, :]]


@functools.lru_cache(maxsize=None)
def _decim_indices(B, N):
    cur, sel = N, []
    with jax.ensure_compile_time_eval():
        key = jax.random.key(1234)
        for i in range(3):
            S = max(1, int(cur * 0.25))
            r = np.asarray(
                jax.random.uniform(jax.random.fold_in(key, i), (B, cur)))
            sel.append(np.argsort(r, axis=1, kind='stable')[:, :S])
            cur = S
    a0, a1, a2 = sel
    i0 = a0
    i1 = np.take_along_axis(i0, a1, axis=1)
    i2 = np.take_along_axis(i1, a2, axis=1)
    return i0.astype(np.int32), i1.astype(np.int32), i2.astype(np.int32)


def kernel(xyz, features, params):
    B, N, _ = xyz.shape
    f32 = jnp.float32
    i0, i1, i2 = _decim_indices(B, N)

    s0, s1, s2 = i0.shape[1], i1.shape[1], i2.shape[1]
    per_b = s0 + s1 + s2
    width = 8
    table = jnp.concatenate(
        [xyz, jnp.transpose(features, (0, 2, 1)),
         jnp.zeros((B, N, width - 6), f32)], axis=2).reshape(B * N, width)
    idx_all = (np.concatenate([i0, i1, i2], axis=1)
               + (np.arange(B, dtype=np.int32) * N)[:, None]).reshape(-1)
    total = B * per_b
    nw = 32
    per_w = total // nw
    n_chunks = 1
    while per_w % n_chunks or per_w // n_chunks > 128:
        n_chunks += 1
    chunk = per_w // n_chunks
    g = _sc_gather(table,
                   jnp.asarray(idx_all.reshape(total // chunk, chunk)),
                   total // chunk, chunk, width)
    g = g.reshape(B, per_b, width)
    qtr = g[:, s0 + s1:, :3]
    fin = g[:, s0 + s1:, 3:6]
    gt = jnp.transpose(g[:, :, :3], (0, 2, 1))
    c0t = gt[:, :, :s0]
    c1t = gt[:, :, s0:s0 + s1]
    c2t = gt[:, :, s0 + s1:]
    ws = _wt(params['embedding'][0])
    for p in params['lfa']:
        w1, b1 = p['enc'][0]
        w1t = w1.T
        ws += [w1t[0:3] + w1t[7:10], w1t[4:7] - w1t[0:3], w1t[3:4], b1[None, :]]
        ws += _wt(p['enc'][1])
        for wb in p['att_mlp']:
            ws += _wt(wb)
        ws += _wt(p['att_conv'])
        for wb in p['out']:
            ws += _wt(wb)
        ws += _wt(p['short'])

    data = [c0t, c1t, c2t, qtr, fin]
    d_specs = [pl.BlockSpec((1,) + d.shape[1:], lambda b: (b, 0, 0)) for d in data]
    w_specs = [pl.BlockSpec(w.shape, lambda b: (0, 0)) for w in ws]

    out = pl.pallas_call(
        _tc_body,
        grid=(B,),
        in_specs=d_specs + w_specs,
        out_specs=pl.BlockSpec((1, _DIMS[-1], _NQ), lambda b: (b, 0, 0)),
        out_shape=jax.ShapeDtypeStruct((B, _DIMS[-1], _NQ), f32),
        interpret=_INTERPRET,
    )(*data, *ws)
    return out
